# speedup vs baseline: 1.0000x; 1.0000x over previous
import jax
import jax.numpy as jnp
from jax.experimental import pallas as pl

_THRESHOLD = 0.5
_NEG_POS = 3
_ALPHA = 0.5


def _cxcy2xy(b):
    return jnp.concatenate([b[:, :2] - b[:, 2:] / 2, b[:, :2] + b[:, 2:] / 2], axis=1)


def _xy2cxcy(b):
    return jnp.concatenate([(b[:, 2:] + b[:, :2]) / 2, b[:, 2:] - b[:, :2]], axis=1)


def _encode_boxes(boxes, d):
    eps = 1e-06
    return jnp.concatenate([(boxes[:, :2] - d[:, :2]) / (d[:, 2:] / 10 + eps),
                            jnp.log(boxes[:, 2:] / d[:, 2:] + eps) * 5], axis=1)


def _find_iou(a, b):
    lt = jnp.maximum(a[:, None, :2], b[None, :, :2])
    rb = jnp.minimum(a[:, None, 2:], b[None, :, 2:])
    wh = jnp.clip(rb - lt, 0.0)
    inter = wh[..., 0] * wh[..., 1]
    area_a = (a[:, 2] - a[:, 0]) * (a[:, 3] - a[:, 1])
    area_b = (b[:, 2] - b[:, 0]) * (b[:, 3] - b[:, 1])
    return inter / (area_a[:, None] + area_b[None, :] - inter)


def kernel(locs_pred, cls_pred, boxes, labels, default_boxes):
    def_xy = _cxcy2xy(default_boxes)
    n_def = default_boxes.shape[0]

    def match_one(boxes_i, labels_i):
        n_obj = boxes_i.shape[0]
        overlap = _find_iou(boxes_i, def_xy)
        overlap_defbox = jnp.max(overlap, axis=0)
        object_defbox = jnp.argmax(overlap, axis=0)
        defbox_object = jnp.argmax(overlap, axis=1)
        object_defbox = object_defbox.at[defbox_object].set(jnp.arange(n_obj))
        overlap_defbox = overlap_defbox.at[defbox_object].set(1.0)
        label_defbox = labels_i[object_defbox]
        label_defbox = jnp.where(overlap_defbox < _THRESHOLD, 0, label_defbox)
        t_locs_i = _encode_boxes(_xy2cxcy(boxes_i[object_defbox]), default_boxes)
        return label_defbox, t_locs_i

    t_classes, t_locs = jax.vmap(match_one)(boxes, labels)

    pos = t_classes != 0
    n_positive = jnp.sum(pos, axis=1)
    n_pos_total = jnp.sum(n_positive)

    d = locs_pred - t_locs
    ad = jnp.abs(d)
    sl1 = jnp.where(ad < 1.0, 0.5 * d * d, ad - 0.5)
    loc_loss = jnp.sum(sl1 * pos[..., None].astype(sl1.dtype)) / jnp.maximum(
        n_pos_total.astype(jnp.float32) * 4.0, 1.0)

    logp = jax.nn.log_softmax(cls_pred, axis=-1)
    all_conf = -jnp.take_along_axis(logp, t_classes[..., None], axis=-1)[..., 0]
    pos_sum = jnp.sum(jnp.where(pos, all_conf, 0.0))
    neg_conf = jnp.where(pos, 0.0, all_conf)
    neg_sorted = -jnp.sort(-neg_conf, axis=1)
    ranks = jnp.arange(n_def)[None, :]
    hard = ranks < (_NEG_POS * n_positive)[:, None]
    neg_sum = jnp.sum(jnp.where(hard, neg_sorted, 0.0))
    conf_loss = (neg_sum + pos_sum) / jnp.maximum(n_pos_total.astype(jnp.float32), 1.0)
    return _ALPHA * loc_loss + conf_loss


# speedup vs baseline: 26.1278x; 26.1276x over previous
import functools

import jax
import jax.numpy as jnp
from jax import lax
from jax.experimental import pallas as pl
from jax.experimental.pallas import tpu as pltpu
from jax.experimental.pallas import tpu_sc as plsc

_THRESHOLD = 0.5
_NEG_POS = 3
_ALPHA = 0.5

_B = 32
_ND = 24564
_NDP = 24576
_NC = 21
_NOBJ = 16


def _tc_body(boxes_ref, gtab_ref, labels_ref, defxy_ref, defct_ref,
             cls_ref, locs_ref, stats_ref, conf_ref, posf_ref):
    f32 = jnp.float32
    bx = boxes_ref[0]
    b_x0, b_y0 = bx[:, 0:1], bx[:, 1:2]
    b_x1, b_y1 = bx[:, 2:3], bx[:, 3:4]
    d_x0 = defxy_ref[0:1, :]
    d_y0 = defxy_ref[1:2, :]
    d_x1 = defxy_ref[2:3, :]
    d_y1 = defxy_ref[3:4, :]

    w = jnp.maximum(jnp.minimum(b_x1, d_x1) - jnp.maximum(b_x0, d_x0), 0.0)
    h = jnp.maximum(jnp.minimum(b_y1, d_y1) - jnp.maximum(b_y0, d_y0), 0.0)
    inter = w * h
    area_a = (b_x1 - b_x0) * (b_y1 - b_y0)
    area_b = (d_x1 - d_x0) * (d_y1 - d_y0)
    iou = inter / (area_a + area_b - inter)

    obj_iota = lax.broadcasted_iota(jnp.int32, (_NOBJ, _ND), 0)
    lane_iota = lax.broadcasted_iota(jnp.int32, (_NOBJ, _ND), 1)

    ovl = jnp.max(iou, axis=0, keepdims=True)
    od = jnp.min(jnp.where(iou == ovl, obj_iota, _NOBJ),
                 axis=0, keepdims=True)
    rowmax = jnp.max(iou, axis=1, keepdims=True)
    dbo = jnp.min(jnp.where(iou == rowmax, lane_iota, _ND),
                  axis=1, keepdims=True)

    feq = dbo == lane_iota
    jf = jnp.max(jnp.where(feq, obj_iota, -1), axis=0, keepdims=True)
    forced = jf >= 0
    od_f = jnp.where(forced, jf, od)
    ovl_f = jnp.where(forced, 1.0, ovl)

    onehot = (od_f == obj_iota).astype(f32)
    dn = (((1,), (0,)), ((), ()))
    tl = lax.dot_general(labels_ref[0], onehot, dn,
                         preferred_element_type=f32)
    tl = jnp.floor(tl + 0.5)
    g = lax.dot_general(gtab_ref[0], onehot, dn,
                        preferred_element_type=f32)

    tcls = jnp.where(ovl_f < _THRESHOLD, 0.0, tl)
    posm = tcls != 0.0

    dcx = defct_ref[0:1, :]
    dcy = defct_ref[1:2, :]
    dw = defct_ref[2:3, :]
    dh = defct_ref[3:4, :]
    eps = 1e-6
    enc_x = (g[0:1, :] - dcx) / (dw / 10 + eps)
    enc_y = (g[1:2, :] - dcy) / (dh / 10 + eps)
    enc_w = jnp.log(g[2:3, :] / dw + eps) * 5
    enc_h = jnp.log(g[3:4, :] / dh + eps) * 5
    tloc = jnp.concatenate([enc_x, enc_y, enc_w, enc_h], axis=0)

    d = locs_ref[0] - tloc
    ad = jnp.abs(d)
    sl1 = jnp.where(ad < 1.0, 0.5 * d * d, ad - 0.5)
    sl1_sum = jnp.sum(jnp.where(posm, sl1, 0.0))

    x = cls_ref[0]
    mx = jnp.max(x, axis=0, keepdims=True)
    lse = mx + jnp.log(jnp.sum(jnp.exp(x - mx), axis=0, keepdims=True))
    cls_iota = lax.broadcasted_iota(jnp.int32, (_NC, _ND), 0)
    xt = jnp.sum(jnp.where(cls_iota == tcls.astype(jnp.int32), x, 0.0),
                 axis=0, keepdims=True)
    aconf = lse - xt

    posf = jnp.where(posm, 1.0, 0.0)
    n_pos = jnp.sum(posf)
    pos_sum = jnp.sum(aconf * posf)

    conf_ref[0, 0, :] = jnp.zeros((_NDP,), f32)
    posf_ref[0, 0, :] = jnp.zeros((_NDP,), f32)
    conf_ref[0, 0, 0:_ND] = aconf[0, :]
    posf_ref[0, 0, 0:_ND] = posf[0, :]

    r = lax.broadcasted_iota(jnp.int32, (1, 128), 1)
    row = jnp.where(r == 0, sl1_sum, 0.0)
    row = jnp.where(r == 1, n_pos, row)
    row = jnp.where(r == 2, pos_sum, row)
    stats_ref[0, :, :] = row


def _tc_pass(boxes, gtab, labels_f, defxy_t, defct_t, cls_t, locs_t):
    grid = (_B,)
    in_specs = [
        pl.BlockSpec((1, _NOBJ, 4), lambda i: (i, 0, 0)),
        pl.BlockSpec((1, 4, _NOBJ), lambda i: (i, 0, 0)),
        pl.BlockSpec((1, 1, _NOBJ), lambda i: (i, 0, 0)),
        pl.BlockSpec((4, _ND), lambda i: (0, 0)),
        pl.BlockSpec((4, _ND), lambda i: (0, 0)),
        pl.BlockSpec((1, _NC, _ND), lambda i: (i, 0, 0)),
        pl.BlockSpec((1, 4, _ND), lambda i: (i, 0, 0)),
    ]
    out_specs = [
        pl.BlockSpec((1, 1, 128), lambda i: (i, 0, 0)),
        pl.BlockSpec((1, 1, _NDP), lambda i: (i, 0, 0)),
        pl.BlockSpec((1, 1, _NDP), lambda i: (i, 0, 0)),
    ]
    out_shape = [
        jax.ShapeDtypeStruct((_B, 1, 128), jnp.float32),
        jax.ShapeDtypeStruct((_B, 1, _NDP), jnp.float32),
        jax.ShapeDtypeStruct((_B, 1, _NDP), jnp.float32),
    ]
    return pl.pallas_call(
        _tc_body, grid=grid, in_specs=in_specs, out_specs=out_specs,
        out_shape=out_shape,
    )(boxes, gtab, labels_f, defxy_t, defct_t, cls_t, locs_t)


def _sc_mine(conf2d, posf2d):
    info = plsc.get_sparse_core_info()
    nc = info.num_cores
    mesh = plsc.VectorSubcoreMesh(core_axis_name="c", subcore_axis_name="s")
    nch = _NDP // 16

    @functools.partial(
        pl.kernel, mesh=mesh,
        out_type=jax.ShapeDtypeStruct((_B, 16), jnp.float32),
        scratch_types=[
            pltpu.VMEM((_NDP,), jnp.float32),
            pltpu.VMEM((_NDP,), jnp.float32),
            pltpu.VMEM((_NDP,), jnp.int32),
            pltpu.VMEM((16,), jnp.float32),
            pltpu.VMEM((16,), jnp.float32),
            pltpu.VMEM((16,), jnp.int32),
        ],
    )
    def body(conf_hbm, posf_hbm, out_hbm, conf_v, posf_v, negi_v, row_v,
             tmpf_v, tmpi_v):
        wid = lax.axis_index("s") * nc + lax.axis_index("c")
        pltpu.sync_copy(conf_hbm.at[wid], conf_v)
        pltpu.sync_copy(posf_hbm.at[wid], posf_v)

        def hsum(vec):
            s = vec[0]
            for j in range(1, 16):
                s = s + vec[j]
            return s

        hsum_f = hsum
        hsum_i = hsum

        def prep(i, car):
            np_acc, ps_acc = car
            c = conf_v[pl.ds(i * 16, 16)]
            p = posf_v[pl.ds(i * 16, 16)]
            neg = c * (1.0 - p)
            negi_v[pl.ds(i * 16, 16)] = lax.bitcast_convert_type(neg, jnp.int32)
            return np_acc + p, ps_acc + c * p

        z = jnp.zeros((16,), jnp.float32)
        np_acc, ps_acc = lax.fori_loop(0, nch, prep, (z, z))
        n_pos = hsum_f(np_acc)
        pos_sum = hsum_f(ps_acc)
        k = _NEG_POS * n_pos.astype(jnp.int32)

        def count_ge(t):
            def cbody(i, acc):
                v = negi_v[pl.ds(i * 16, 16)]
                return acc + jnp.where(v >= t, 1, 0)
            cnt = lax.fori_loop(0, nch, cbody,
                                jnp.zeros((16,), jnp.int32))
            return hsum_i(cnt)

        def sbody(b, m):
            cand = m | (jnp.int32(1) << (jnp.int32(30) - b))
            return jnp.where(count_ge(cand) >= k, cand, m)

        m = lax.fori_loop(0, 31, sbody, jnp.int32(0))

        def fbody(i, car):
            s_acc, c_acc = car
            vi = negi_v[pl.ds(i * 16, 16)]
            vf = lax.bitcast_convert_type(vi, jnp.float32)
            gt = vi > m
            return (s_acc + jnp.where(gt, vf, 0.0),
                    c_acc + jnp.where(gt, 1, 0))

        s_acc, c_acc = lax.fori_loop(
            0, nch, fbody,
            (jnp.zeros((16,), jnp.float32), jnp.zeros((16,), jnp.int32)))
        s = hsum_f(s_acc)
        c_strict = hsum_i(c_acc)
        m_f_vec = lax.bitcast_convert_type(
            jnp.broadcast_to(m, (16,)), jnp.float32)
        fill_vec = (k - c_strict).astype(jnp.float32) * m_f_vec
        neg_sum_vec = jnp.where(k > 0, s + fill_vec, 0.0)

        r = lax.broadcasted_iota(jnp.int32, (16,), 0)
        row = jnp.where(r == 0, neg_sum_vec, 0.0)
        row = jnp.where(r == 1, pos_sum, row)
        row = jnp.where(r == 2, n_pos, row)
        row_v[...] = row
        pltpu.sync_copy(row_v, out_hbm.at[wid])

    return body(conf2d, posf2d)


def kernel(locs_pred, cls_pred, boxes, labels, default_boxes):
    f32 = jnp.float32
    cls_t = jnp.transpose(cls_pred, (0, 2, 1))
    locs_t = jnp.transpose(locs_pred, (0, 2, 1))
    d = default_boxes
    defxy = jnp.concatenate([d[:, :2] - d[:, 2:] / 2,
                             d[:, :2] + d[:, 2:] / 2], axis=1)
    defxy_t = jnp.transpose(defxy, (1, 0))
    defct_t = jnp.transpose(d, (1, 0))
    b0, b1 = boxes[..., 0], boxes[..., 1]
    b2, b3 = boxes[..., 2], boxes[..., 3]
    gtab = jnp.stack([(b0 + b2) / 2, (b1 + b3) / 2, b2 - b0, b3 - b1],
                     axis=1)
    labels_f = labels.astype(f32)[:, None, :]

    stats, conf, posf = _tc_pass(boxes, gtab, labels_f, defxy_t, defct_t,
                                 cls_t, locs_t)

    sc_out = _sc_mine(conf.reshape(_B, _NDP), posf.reshape(_B, _NDP))

    sl1_total = jnp.sum(stats[:, 0, 0])
    n_pos_total = jnp.sum(stats[:, 0, 1])
    pos_sum = jnp.sum(stats[:, 0, 2])
    neg_sum = jnp.sum(sc_out[:, 0])
    loc_loss = sl1_total / jnp.maximum(n_pos_total * 4.0, 1.0)
    conf_loss = (neg_sum + pos_sum) / jnp.maximum(n_pos_total, 1.0)
    return _ALPHA * loc_loss + conf_loss


# speedup vs baseline: 39.5776x; 1.5148x over previous
import functools

import jax
import jax.numpy as jnp
from jax import lax
from jax.experimental import pallas as pl
from jax.experimental.pallas import tpu as pltpu
from jax.experimental.pallas import tpu_sc as plsc

_THRESHOLD = 0.5
_NEG_POS = 3
_ALPHA = 0.5

_B = 32
_ND = 24564
_NDP = 24576
_NC = 21
_NOBJ = 16
_UNROLL = 8


def _tc_body(boxes_ref, gtab_ref, labels_ref, defxy_ref, defct_ref,
             cls_ref, locs_ref, stats_ref, cp_ref):
    f32 = jnp.float32
    bx = boxes_ref[0]
    b_x0, b_y0 = bx[:, 0:1], bx[:, 1:2]
    b_x1, b_y1 = bx[:, 2:3], bx[:, 3:4]
    d_x0 = defxy_ref[0:1, :]
    d_y0 = defxy_ref[1:2, :]
    d_x1 = defxy_ref[2:3, :]
    d_y1 = defxy_ref[3:4, :]

    w = jnp.maximum(jnp.minimum(b_x1, d_x1) - jnp.maximum(b_x0, d_x0), 0.0)
    h = jnp.maximum(jnp.minimum(b_y1, d_y1) - jnp.maximum(b_y0, d_y0), 0.0)
    inter = w * h
    area_a = (b_x1 - b_x0) * (b_y1 - b_y0)
    area_b = (d_x1 - d_x0) * (d_y1 - d_y0)
    iou = inter / (area_a + area_b - inter)

    obj_iota = lax.broadcasted_iota(jnp.int32, (_NOBJ, _ND), 0)
    lane_iota = lax.broadcasted_iota(jnp.int32, (_NOBJ, _ND), 1)

    ovl = jnp.max(iou, axis=0, keepdims=True)
    od = jnp.min(jnp.where(iou == ovl, obj_iota, _NOBJ),
                 axis=0, keepdims=True)
    rowmax = jnp.max(iou, axis=1, keepdims=True)
    dbo = jnp.min(jnp.where(iou == rowmax, lane_iota, _ND),
                  axis=1, keepdims=True)

    feq = dbo == lane_iota
    jf = jnp.max(jnp.where(feq, obj_iota, -1), axis=0, keepdims=True)
    forced = jf >= 0
    od_f = jnp.where(forced, jf, od)
    ovl_f = jnp.where(forced, 1.0, ovl)

    onehot = (od_f == obj_iota).astype(f32)
    dn = (((1,), (0,)), ((), ()))
    tl = lax.dot_general(labels_ref[0], onehot, dn,
                         preferred_element_type=f32)
    tl = jnp.floor(tl + 0.5)
    g = lax.dot_general(gtab_ref[0], onehot, dn,
                        preferred_element_type=f32)

    tcls = jnp.where(ovl_f < _THRESHOLD, 0.0, tl)
    posm = tcls != 0.0

    dcx = defct_ref[0:1, :]
    dcy = defct_ref[1:2, :]
    dw = defct_ref[2:3, :]
    dh = defct_ref[3:4, :]
    eps = 1e-6
    enc_x = (g[0:1, :] - dcx) / (dw / 10 + eps)
    enc_y = (g[1:2, :] - dcy) / (dh / 10 + eps)
    enc_w = jnp.log(g[2:3, :] / dw + eps) * 5
    enc_h = jnp.log(g[3:4, :] / dh + eps) * 5
    tloc = jnp.concatenate([enc_x, enc_y, enc_w, enc_h], axis=0)

    d = locs_ref[0] - tloc
    ad = jnp.abs(d)
    sl1 = jnp.where(ad < 1.0, 0.5 * d * d, ad - 0.5)
    sl1_sum = jnp.sum(jnp.where(posm, sl1, 0.0))

    x = cls_ref[0]
    mx = jnp.max(x, axis=0, keepdims=True)
    lse = mx + jnp.log(jnp.sum(jnp.exp(x - mx), axis=0, keepdims=True))
    cls_iota = lax.broadcasted_iota(jnp.int32, (_NC, _ND), 0)
    xt = jnp.sum(jnp.where(cls_iota == tcls.astype(jnp.int32), x, 0.0),
                 axis=0, keepdims=True)
    aconf = lse - xt

    posf = jnp.where(posm, 1.0, 0.0)
    n_pos = jnp.sum(posf)
    pos_sum = jnp.sum(aconf * posf)

    cp_ref[...] = jnp.zeros((8, _NDP), f32)
    cp_ref[0, 0:_ND] = aconf[0, :]
    cp_ref[1, 0:_ND] = posf[0, :]

    r = lax.broadcasted_iota(jnp.int32, (1, 128), 1)
    row = jnp.where(r == 0, sl1_sum, 0.0)
    row = jnp.where(r == 1, n_pos, row)
    row = jnp.where(r == 2, pos_sum, row)
    stats_ref[0, :, :] = row


def _tc_pass(boxes, gtab, labels_f, defxy_t, defct_t, cls_t, locs_t):
    grid = (_B,)
    in_specs = [
        pl.BlockSpec((1, _NOBJ, 4), lambda i: (i, 0, 0)),
        pl.BlockSpec((1, 4, _NOBJ), lambda i: (i, 0, 0)),
        pl.BlockSpec((1, 1, _NOBJ), lambda i: (i, 0, 0)),
        pl.BlockSpec((4, _ND), lambda i: (0, 0)),
        pl.BlockSpec((4, _ND), lambda i: (0, 0)),
        pl.BlockSpec((1, _NC, _ND), lambda i: (i, 0, 0)),
        pl.BlockSpec((1, 4, _ND), lambda i: (i, 0, 0)),
    ]
    out_specs = [
        pl.BlockSpec((1, 1, 128), lambda i: (i, 0, 0)),
        pl.BlockSpec((8, _NDP), lambda i: (i, 0)),
    ]
    out_shape = [
        jax.ShapeDtypeStruct((_B, 1, 128), jnp.float32),
        jax.ShapeDtypeStruct((_B * 8, _NDP), jnp.float32),
    ]
    return pl.pallas_call(
        _tc_body, grid=grid, in_specs=in_specs, out_specs=out_specs,
        out_shape=out_shape,
    )(boxes, gtab, labels_f, defxy_t, defct_t, cls_t, locs_t)


def _sc_mine(cp2d):
    info = plsc.get_sparse_core_info()
    nc = info.num_cores
    mesh = plsc.VectorSubcoreMesh(core_axis_name="c", subcore_axis_name="s")
    nch = _NDP // (16 * _UNROLL)

    @functools.partial(
        pl.kernel, mesh=mesh,
        out_type=jax.ShapeDtypeStruct((_B, 16), jnp.float32),
        scratch_types=[
            pltpu.VMEM((_NDP,), jnp.float32),
            pltpu.VMEM((_NDP,), jnp.float32),
            pltpu.VMEM((_NDP,), jnp.int32),
            pltpu.VMEM((16,), jnp.float32),
            pltpu.VMEM((16,), jnp.float32),
            pltpu.VMEM((16,), jnp.int32),
        ],
    )
    def body(cp_hbm, out_hbm, conf_v, posf_v, negi_v, row_v,
             tmpf_v, tmpi_v):
        wid = lax.axis_index("s") * nc + lax.axis_index("c")
        pltpu.sync_copy(cp_hbm.at[8 * wid], conf_v)
        pltpu.sync_copy(cp_hbm.at[8 * wid + 1], posf_v)

        def hsum(vec):
            s = vec[0]
            for j in range(1, 16):
                s = s + vec[j]
            return s

        hsum_f = hsum
        hsum_i = hsum

        def prep(i, car):
            np_acc, ps_acc = car
            for j in range(_UNROLL):
                o = i * (16 * _UNROLL) + j * 16
                c = conf_v[pl.ds(o, 16)]
                p = posf_v[pl.ds(o, 16)]
                neg = c * (1.0 - p)
                negi_v[pl.ds(o, 16)] = lax.bitcast_convert_type(neg, jnp.int32)
                np_acc = np_acc + p
                ps_acc = ps_acc + c * p
            return np_acc, ps_acc

        z = jnp.zeros((16,), jnp.float32)
        np_acc, ps_acc = lax.fori_loop(0, nch, prep, (z, z))
        n_pos = hsum_f(np_acc)
        pos_sum = hsum_f(ps_acc)
        k = _NEG_POS * n_pos.astype(jnp.int32)

        def count_ge(t):
            def cbody(i, acc):
                for j in range(_UNROLL):
                    v = negi_v[pl.ds(i * (16 * _UNROLL) + j * 16, 16)]
                    acc = acc + jnp.where(v >= t, 1, 0)
                return acc
            cnt = lax.fori_loop(0, nch, cbody,
                                jnp.zeros((16,), jnp.int32))
            return hsum_i(cnt)

        def sbody(b, m):
            cand = m | (jnp.int32(1) << (jnp.int32(30) - b))
            return jnp.where(count_ge(cand) >= k, cand, m)

        m = lax.fori_loop(0, 31, sbody, jnp.int32(0))

        def fbody(i, car):
            s_acc, c_acc = car
            for j in range(_UNROLL):
                vi = negi_v[pl.ds(i * (16 * _UNROLL) + j * 16, 16)]
                vf = lax.bitcast_convert_type(vi, jnp.float32)
                gt = vi > m
                s_acc = s_acc + jnp.where(gt, vf, 0.0)
                c_acc = c_acc + jnp.where(gt, 1, 0)
            return s_acc, c_acc

        s_acc, c_acc = lax.fori_loop(
            0, nch, fbody,
            (jnp.zeros((16,), jnp.float32), jnp.zeros((16,), jnp.int32)))
        s = hsum_f(s_acc)
        c_strict = hsum_i(c_acc)
        m_f_vec = lax.bitcast_convert_type(
            jnp.broadcast_to(m, (16,)), jnp.float32)
        fill_vec = (k - c_strict).astype(jnp.float32) * m_f_vec
        neg_sum_vec = jnp.where(k > 0, s + fill_vec, 0.0)

        r = lax.broadcasted_iota(jnp.int32, (16,), 0)
        row = jnp.where(r == 0, neg_sum_vec, 0.0)
        row = jnp.where(r == 1, pos_sum, row)
        row = jnp.where(r == 2, n_pos, row)
        row_v[...] = row
        pltpu.sync_copy(row_v, out_hbm.at[wid])

    return body(cp2d)


def kernel(locs_pred, cls_pred, boxes, labels, default_boxes):
    f32 = jnp.float32
    cls_t = jnp.transpose(cls_pred, (0, 2, 1))
    locs_t = jnp.transpose(locs_pred, (0, 2, 1))
    d = default_boxes
    defxy = jnp.concatenate([d[:, :2] - d[:, 2:] / 2,
                             d[:, :2] + d[:, 2:] / 2], axis=1)
    defxy_t = jnp.transpose(defxy, (1, 0))
    defct_t = jnp.transpose(d, (1, 0))
    b0, b1 = boxes[..., 0], boxes[..., 1]
    b2, b3 = boxes[..., 2], boxes[..., 3]
    gtab = jnp.stack([(b0 + b2) / 2, (b1 + b3) / 2, b2 - b0, b3 - b1],
                     axis=1)
    labels_f = labels.astype(f32)[:, None, :]

    stats, cp2d = _tc_pass(boxes, gtab, labels_f, defxy_t, defct_t,
                           cls_t, locs_t)

    sc_out = _sc_mine(cp2d)

    sl1_total = jnp.sum(stats[:, 0, 0])
    n_pos_total = jnp.sum(stats[:, 0, 1])
    pos_sum = jnp.sum(stats[:, 0, 2])
    neg_sum = jnp.sum(sc_out[:, 0])
    loc_loss = sl1_total / jnp.maximum(n_pos_total * 4.0, 1.0)
    conf_loss = (neg_sum + pos_sum) / jnp.maximum(n_pos_total, 1.0)
    return _ALPHA * loc_loss + conf_loss
